# baseline (speedup 1.0000x reference)
import functools

import jax
import jax.numpy as jnp
import numpy as np
from jax import lax
from jax.experimental import pallas as pl
from jax.experimental.pallas import tpu as pltpu
from jax.experimental.pallas import tpu_sc as plsc

_VOCAB = 30522
_DIM = 768
_MAX_LEN = 64
_BATCH = 4096
_SEQ = 64

_N = _BATCH * _SEQ
_NC = 2
_NS = 16
_NW = _NC * _NS
_RPW = _N // _NW
_CH = 16
_NBUF = 6
_AHEAD = 3
_HROWS = _RPW // 2
_HCH = _HROWS // _CH
_LANES = 16
_COLS = _DIM // _LANES


def _sincos_pos(length, dim):
    pos = np.arange(length, dtype=np.float32)[:, None]
    i = np.arange(dim // 2, dtype=np.float32)[None, :]
    angle = pos / np.power(10000.0, 2.0 * i / dim)
    return np.concatenate([np.sin(angle), np.cos(angle)], axis=-1)


_mesh = plsc.VectorSubcoreMesh(
    core_axis_name="c", subcore_axis_name="s", num_cores=_NC, num_subcores=_NS
)


@functools.partial(
    pl.kernel,
    out_type=jax.ShapeDtypeStruct((_N, _DIM), jnp.float32),
    mesh=_mesh,
    scratch_types=[
        pltpu.VMEM((_HROWS,), jnp.int32),
        pltpu.VMEM((_MAX_LEN, _DIM), jnp.float32),
        pltpu.VMEM((_NBUF, _CH, _DIM), jnp.float32),
        pltpu.SemaphoreType.DMA((_NBUF,)),
        pltpu.SemaphoreType.DMA((_NBUF,)),
    ],
)
def _embed(x_hbm, wte_hbm, pos_hbm, out_hbm, idx_v, pos_v, rows_v, gsem, osem):
    wid = lax.axis_index("s") * _NC + lax.axis_index("c")
    base = wid * _RPW
    pltpu.sync_copy(pos_hbm, pos_v)

    def g_desc(c, b):
        return pltpu.make_async_copy(
            wte_hbm.at[idx_v.at[pl.ds(c * _CH, _CH)]],
            rows_v.at[b],
            gsem.at[b],
        )

    def o_desc(rbase, c, b):
        return pltpu.make_async_copy(
            rows_v.at[b],
            out_hbm.at[pl.ds(rbase + c * _CH, _CH)],
            osem.at[b],
        )

    def compute(c, b):
        s0 = lax.rem(c * _CH, _MAX_LEN)

        def row(r, _):
            for cc in range(_COLS):
                sl = pl.ds(cc * _LANES, _LANES)
                p = pos_v[s0 + r, sl]
                plsc.addupdate(rows_v.at[b, r, sl], p)
            return 0

        lax.fori_loop(0, _CH, row, 0)

    def run_pass(rbase):
        def step(c, skip_owait=False, issue_ahead=True):
            b = lax.rem(c, _NBUF)
            g_desc(c, b).wait()
            compute(c, b)
            o_desc(rbase, c, b).start()
            if issue_ahead:
                f = c + _AHEAD
                bf = lax.rem(f, _NBUF)
                if not skip_owait:
                    o_desc(rbase, c, bf).wait()
                g_desc(f, bf).start()

        for k in range(_AHEAD):
            g_desc(jnp.int32(k), jnp.int32(k)).start()
        for k in range(_NBUF - _AHEAD):
            step(jnp.int32(k), skip_owait=True)

        def body(c, _):
            step(c)
            return 0

        lax.fori_loop(_NBUF - _AHEAD, _HCH - _AHEAD, body, 0)

        for k in range(_HCH - _AHEAD, _HCH):
            step(jnp.int32(k), issue_ahead=False)
        for k in range(_HCH - _NBUF, _HCH):
            o_desc(rbase, jnp.int32(k), jnp.int32(k % _NBUF)).wait()

    pltpu.sync_copy(x_hbm.at[pl.ds(base, _HROWS)], idx_v)
    run_pass(base)
    pltpu.sync_copy(x_hbm.at[pl.ds(base + _HROWS, _HROWS)], idx_v)
    run_pass(base + _HROWS)


def kernel(x, wte):
    pos = jnp.asarray(_sincos_pos(_MAX_LEN, _DIM), dtype=jnp.float32)
    xf = jnp.asarray(x, jnp.int32).reshape(_N)
    out = _embed(xf, wte, pos)
    return out.reshape(_BATCH, _SEQ, _DIM)


# speedup vs baseline: 2.1874x; 2.1874x over previous
import functools

import jax
import jax.numpy as jnp
import numpy as np
from jax import lax
from jax.experimental import pallas as pl
from jax.experimental.pallas import tpu as pltpu
from jax.experimental.pallas import tpu_sc as plsc

_VOCAB = 30522
_DIM = 768
_MAX_LEN = 64
_BATCH = 4096
_SEQ = 64

_N = _BATCH * _SEQ
_NC = 2
_NS = 16
_NW = _NC * _NS
_RPW = _N // _NW
_CH = 8
_NBUF = 8
_NCH = _RPW // _CH
_LANES = 16
_COLS = _DIM // _LANES


def _sincos_pos(length, dim):
    pos = np.arange(length, dtype=np.float32)[:, None]
    i = np.arange(dim // 2, dtype=np.float32)[None, :]
    angle = pos / np.power(10000.0, 2.0 * i / dim)
    return np.concatenate([np.sin(angle), np.cos(angle)], axis=-1)


_mesh = plsc.VectorSubcoreMesh(
    core_axis_name="c", subcore_axis_name="s", num_cores=_NC, num_subcores=_NS
)


@functools.partial(
    pl.kernel,
    out_type=jax.ShapeDtypeStruct((_N, _DIM), jnp.float32),
    mesh=_mesh,
    scratch_types=[
        pltpu.VMEM((_RPW,), jnp.int32),
        pltpu.VMEM((_MAX_LEN, _DIM), jnp.float32),
        pltpu.VMEM((_NBUF, _CH, _DIM), jnp.float32),
        pltpu.SemaphoreType.DMA((_NBUF,)),
        pltpu.SemaphoreType.DMA((_NBUF,)),
    ],
)
def _embed(x_hbm, wte_hbm, pos_hbm, out_hbm, idx_v, pos_v, rows_v, gsem, osem):
    wid = lax.axis_index("s") * _NC + lax.axis_index("c")
    base = wid * _RPW
    pltpu.sync_copy(x_hbm.at[pl.ds(base, _RPW)], idx_v)
    pltpu.sync_copy(pos_hbm, pos_v)

    def g_desc(c, b):
        return pltpu.make_async_copy(
            wte_hbm.at[idx_v.at[pl.ds(c * _CH, _CH)]],
            rows_v.at[b],
            gsem.at[b],
        )

    def o_desc(c, b):
        return pltpu.make_async_copy(
            rows_v.at[b],
            out_hbm.at[pl.ds(base + c * _CH, _CH)],
            osem.at[b],
        )

    def compute(b):
        def row(r, _, b=b):
            for cc in range(_COLS):
                sl = pl.ds(cc * _LANES, _LANES)
                p = pos_v[b * _CH + r, sl]
                plsc.addupdate(rows_v.at[b, r, sl], p)
            return 0

        lax.fori_loop(0, _CH, row, 0)

    def step(c, b, skip_owait=False, issue_ahead=True):
        g_desc(c, b).wait()
        compute(b)
        o_desc(c, b).start()
        if issue_ahead:
            f = c + _NBUF // 2
            bf = (b + _NBUF // 2) % _NBUF
            if not skip_owait:
                o_desc(c, bf).wait()
            g_desc(f, bf).start()

    for k in range(_NBUF // 2):
        g_desc(k, k).start()

    for k in range(_NBUF // 2):
        step(k, k, skip_owait=True)
    for k in range(_NBUF // 2, _NBUF):
        step(k, k)

    def body(i, _):
        for b in range(_NBUF):
            step(_NBUF * i + b, b)
        return 0

    lax.fori_loop(1, _NCH // _NBUF - 1, body, 0)

    for k in range(_NBUF // 2):
        step(_NCH - _NBUF + k, k)
    for k in range(_NBUF // 2, _NBUF):
        step(_NCH - _NBUF + k, k, issue_ahead=False)

    for b in range(_NBUF):
        o_desc(_NCH - _NBUF + b, b).wait()


def kernel(x, wte):
    pos = jnp.asarray(_sincos_pos(_MAX_LEN, _DIM), dtype=jnp.float32)
    xf = jnp.asarray(x, jnp.int32).reshape(_N)
    out = _embed(xf, wte, pos)
    return out.reshape(_BATCH, _SEQ, _DIM)
